# Initial kernel scaffold; baseline (speedup 1.0000x reference)
import jax
import jax.numpy as jnp
from jax.experimental import pallas as pl


def kernel(bond_representations, atomic_features, atom_bond_neighbors, W_o, b_o):
    raise NotImplementedError("write your pallas kernel here")


# speedup vs baseline: 3.5975x; 3.5975x over previous
import functools

import jax
import jax.numpy as jnp
from jax import lax
from jax.experimental import pallas as pl
from jax.experimental.pallas import tpu as pltpu
from jax.experimental.pallas import tpu_sc as plsc

_B = 8
_NB = 32768
_NA = 8192
_K = 16
_DB = 64
_DA = 64
_H = 128

_NC, _NS = 2, 16
_NW = _NC * _NS
_APW = (_B * _NA) // _NW
_RPG = 128
_APG = _RPG // _K
_STEPS = (_APW * _K) // _RPG
_FLUSH = 16


def _sc_body(table, idxm, out, idx_v, buf0, buf1, out_v, sem0, sem1):
    w = lax.axis_index("s") * _NC + lax.axis_index("c")
    pltpu.sync_copy(idxm.at[pl.ds(w * _STEPS, _STEPS)], idx_v)

    bufs = (buf0, buf1)
    sems = (sem0, sem1)
    pltpu.async_copy(table.at[idx_v.at[0]], buf0, sem0)
    pltpu.async_copy(table.at[idx_v.at[1]], buf1, sem1)
    out_base = w * _APW

    def reduce_step(s, buf):
        row0 = (s % _FLUSH) * _APG
        for k in range(_APG):
            for d in range(_DB // 16):
                acc = buf[k * _K, pl.ds(d * 16, 16)]
                for j in range(1, _K):
                    acc = acc + buf[k * _K + j, pl.ds(d * 16, 16)]
                out_v[row0 + k, pl.ds(d * 16, 16)] = acc

    def body(t, carry):
        for p in range(2):
            s = 2 * t + p
            buf, sem = bufs[p], sems[p]
            pltpu.make_async_copy(table.at[idx_v.at[0]], buf, sem).wait()
            reduce_step(s, buf)
            nxt = jnp.minimum(s + 2, _STEPS - 1)
            pltpu.async_copy(table.at[idx_v.at[nxt]], buf, sem)

        @pl.when((t + 1) % (_FLUSH // 2) == 0)
        def _flush():
            blk = (2 * t + 1) // _FLUSH
            dst = out.at[pl.ds(out_base + blk * (_FLUSH * _APG), _FLUSH * _APG)]
            pltpu.sync_copy(out_v, dst)

        return carry

    lax.fori_loop(0, _STEPS // 2, body, None)
    pltpu.make_async_copy(table.at[idx_v.at[0]], buf0, sem0).wait()
    pltpu.make_async_copy(table.at[idx_v.at[0]], buf1, sem1).wait()


@functools.lru_cache(maxsize=1)
def _sc_gather_sum():
    return pl.kernel(
        _sc_body,
        out_type=jax.ShapeDtypeStruct((_B * _NA, _DB), jnp.float32),
        mesh=plsc.VectorSubcoreMesh(
            core_axis_name="c", subcore_axis_name="s",
            num_cores=_NC, num_subcores=_NS,
        ),
        scratch_types=[
            pltpu.VMEM((_STEPS, _RPG), jnp.int32),
            pltpu.VMEM((_RPG, _DB), jnp.float32),
            pltpu.VMEM((_RPG, _DB), jnp.float32),
            pltpu.VMEM((_FLUSH * _APG, _DB), jnp.float32),
            pltpu.SemaphoreType.DMA,
            pltpu.SemaphoreType.DMA,
        ],
        compiler_params=pltpu.CompilerParams(use_tc_tiling_on_sc=False),
    )


def _tc_body(af_ref, msg_ref, w1_ref, w2_ref, b_ref, out_ref):
    af = af_ref[0]
    msg = msg_ref[0]
    h = jnp.dot(af, w1_ref[...], preferred_element_type=jnp.float32)
    h = h + jnp.dot(msg, w2_ref[...], preferred_element_type=jnp.float32)
    h = jnp.maximum(h + b_ref[...], 0.0)
    i = pl.program_id(0)
    out_ref[pl.ds(i, 1), :] = jnp.sum(h, axis=0, keepdims=True)


@functools.partial(jax.jit, static_argnums=())
def _tc_readout(af, msg, w1, w2, b):
    return pl.pallas_call(
        _tc_body,
        grid=(_B,),
        in_specs=[
            pl.BlockSpec((1, _NA, _DA), lambda i: (i, 0, 0)),
            pl.BlockSpec((1, _NA, _DB), lambda i: (i, 0, 0)),
            pl.BlockSpec((_DA, _H), lambda i: (0, 0)),
            pl.BlockSpec((_DB, _H), lambda i: (0, 0)),
            pl.BlockSpec((1, _H), lambda i: (0, 0)),
        ],
        out_specs=pl.BlockSpec((_B, _H), lambda i: (0, 0)),
        out_shape=jax.ShapeDtypeStruct((_B, _H), jnp.float32),
    )(af, msg, w1, w2, b)


def kernel(bond_representations, atomic_features, atom_bond_neighbors, W_o, b_o):
    table = bond_representations[0].reshape(_B * _NB, _DB)
    offs = (jnp.arange(_B, dtype=jnp.int32) * _NB)[:, None, None]
    idx = (atom_bond_neighbors.astype(jnp.int32) + offs).reshape(-1, _RPG)
    msg = _sc_gather_sum()(table, idx)
    return _tc_readout(
        atomic_features,
        msg.reshape(_B, _NA, _DB),
        W_o[:_DA],
        W_o[_DA:],
        b_o.reshape(1, _H),
    )
